# baseline (speedup 1.0000x reference)
import math

import jax
import jax.numpy as jnp
from jax import lax
from jax.experimental import pallas as pl
from jax.experimental.pallas import tpu as pltpu
from jax.experimental.pallas import tpu_sc as plsc

_H = 512
_W = 512
_B = 2
_IMH = 2048
_IMW = 2048
_NPOS = _B * _H * _W
_NPIX = _IMH * _IMW
_NLINE = _NPIX // 4
_HW = _H * _W

_NW = 32
_PERW = _NPOS // _NW
_P = 512
_CHUNKS = _PERW // _P

_PI = math.pi
_RB = 32
_BLK = _RB * _W


def _coord_body(in_ref, m0, m1, m2, m3, l0, l1, w0, w1, w2, w3):
    m_refs = [m0, m1, m2, m3]
    l_refs = [l0, l1]
    w_refs = [w0, w1, w2, w3]
    a0 = -in_ref[0, 0]
    a1 = -in_ref[0, 1]
    a2 = -in_ref[0, 2]
    a3 = -in_ref[0, 3]
    ty = jnp.arctan2(a1, a0)
    tx = jnp.arctan2(a3, a2)

    def to_px(t):
        ic = (t + _PI) / (2.0 * _PI)
        ic = -1.0 + 2.0 * ic
        return (ic + 1.0) * 0.5 * (_IMW - 1)

    x = to_px(tx)
    y = to_px(ty)
    x0f = jnp.floor(x)
    y0f = jnp.floor(y)
    wx1 = x - x0f
    wx0 = 1.0 - wx1
    wy1 = y - y0f
    wy0 = 1.0 - wy1
    x0 = jnp.clip(x0f.astype(jnp.int32), 0, _IMW - 1)
    y0 = jnp.clip(y0f.astype(jnp.int32), 0, _IMH - 1)
    x1 = jnp.minimum(x0 + 1, _IMW - 1)
    y1 = jnp.minimum(y0 + 1, _IMH - 1)
    pix = [y0 * _IMW + x0, y0 * _IMW + x1, y1 * _IMW + x0, y1 * _IMW + x1]
    wsv = [wx0 * wy0, wx1 * wy0, wx0 * wy1, wx1 * wy1]
    for c in range(4):
        m_refs[c][...] = (pix[c] >> 2).reshape(_BLK)
        w_refs[c][...] = wsv[c].reshape(_BLK)
    l_refs[0][...] = ((x0 & 3) * 4).reshape(_BLK)
    l_refs[1][...] = ((x1 & 3) * 4).reshape(_BLK)


_spec1d = pl.BlockSpec((_BLK,), lambda b, r: (b * (_H // _RB) + r,))
_coord_call = pl.pallas_call(
    _coord_body,
    grid=(_B, _H // _RB),
    in_specs=[pl.BlockSpec((1, 4, _RB, _W), lambda b, r: (b, 0, r, 0))],
    out_specs=[_spec1d] * 10,
    out_shape=[jax.ShapeDtypeStruct((_NPOS,), jnp.int32)] * 6
              + [jax.ShapeDtypeStruct((_NPOS,), jnp.float32)] * 4,
)


def _sc_body(tab, m0, m1, m2, m3, l0, l1, w0, w1, w2, w3, out,
             mi_v0, mi_v1, li_v0, li_v1, w_v0, w_v1,
             La0, Lb0, Lc0, Ld0, La1, Lb1, Lc1, Ld1,
             o_v0, o_v1,
             isem, gsem, osem):
    m_args = [m0, m1, m2, m3]
    l_args = [l0, l1]
    w_args = [w0, w1, w2, w3]
    mi_bufs = [mi_v0, mi_v1]
    li_bufs = [li_v0, li_v1]
    w_bufs = [w_v0, w_v1]
    L_bufs = [[La0, Lb0, Lc0, Ld0], [La1, Lb1, Lc1, Ld1]]
    o_bufs = [o_v0, o_v1]
    wid = lax.axis_index("s") * 2 + lax.axis_index("c")
    b = wid // 16
    lane = lax.iota(jnp.int32, 16)

    def issue_idxw(t, k):
        base = wid * _PERW + t * _P
        hs = []
        for c in range(4):
            hs.append(pltpu.async_copy(m_args[c].at[pl.ds(base, _P)],
                                       mi_bufs[k].at[pl.ds(c * _P, _P)],
                                       isem))
            hs.append(pltpu.async_copy(w_args[c].at[pl.ds(base, _P)],
                                       w_bufs[k].at[pl.ds(c * _P, _P)],
                                       isem))
        for c in range(2):
            hs.append(pltpu.async_copy(l_args[c].at[pl.ds(base, _P)],
                                       li_bufs[k].at[pl.ds(c * _P, _P)],
                                       isem))
        return hs

    def issue_gathers(k):
        hs = []
        for c in range(4):
            hs.append(pltpu.async_copy(
                tab.at[mi_bufs[k].at[pl.ds(c * _P, _P)]],
                L_bufs[k][c],
                gsem))
        return hs

    def combine(k):
        w_v, li_v, out_v = w_bufs[k], li_bufs[k], o_bufs[k]
        Ls = L_bufs[k]

        def j_body(j, carry2):
            off = j * 16
            posv = lane + off
            lv = [li_v[pl.ds(0 * _P + off, 16)], li_v[pl.ds(1 * _P + off, 16)]]
            wvs = [w_v[pl.ds(c * _P + off, 16)] for c in range(4)]
            for ch in range(3):
                acc = None
                for c in range(4):
                    lanev = lv[c & 1] + ch
                    val = plsc.load_gather(Ls[c], [posv, lanev])
                    term = wvs[c] * val
                    acc = term if acc is None else acc + term
                out_v[pl.ds(ch * _P + off, 16)] = acc
            return carry2

        lax.fori_loop(0, _P // 16, j_body, 0, unroll=2)

    def issue_outwrite(t, k):
        base = wid * _PERW + t * _P
        inb = base - b * _HW
        hs = []
        for ch in range(3):
            hs.append(pltpu.async_copy(
                o_bufs[k].at[pl.ds(ch * _P, _P)],
                out.at[pl.ds((b * 3 + ch) * _HW + inb, _P)],
                osem))
        return hs

    h_iw = [None] * (_CHUNKS + 2)
    h_g = [None] * _CHUNKS
    h_o = [None] * _CHUNKS

    h_iw[0] = issue_idxw(0, 0)
    for h in h_iw[0]:
        h.wait()
    h_g[0] = issue_gathers(0)
    h_iw[1] = issue_idxw(1, 1)

    for t in range(_CHUNKS):
        k = t % 2
        for h in h_g[t]:
            h.wait()
        if t + 1 < _CHUNKS:
            for h in h_iw[t + 1]:
                h.wait()
            h_g[t + 1] = issue_gathers(1 - k)
        if t >= 2:
            for h in h_o[t - 2]:
                h.wait()
        combine(k)
        h_o[t] = issue_outwrite(t, k)
        if t + 2 < _CHUNKS:
            h_iw[t + 2] = issue_idxw(t + 2, k)
    for h in h_o[_CHUNKS - 2]:
        h.wait()
    for h in h_o[_CHUNKS - 1]:
        h.wait()


def _sc_call(tab, ms, ls, ws):
    mesh = plsc.VectorSubcoreMesh(core_axis_name="c", subcore_axis_name="s")
    f = pl.kernel(
        _sc_body,
        out_type=jax.ShapeDtypeStruct((_B * 3 * _HW,), jnp.float32),
        mesh=mesh,
        compiler_params=pltpu.CompilerParams(needs_layout_passes=False,
                                             use_tc_tiling_on_sc=False),
        scratch_types=(
            [pltpu.VMEM((4 * _P,), jnp.int32)] * 2
            + [pltpu.VMEM((2 * _P,), jnp.int32)] * 2
            + [pltpu.VMEM((4 * _P,), jnp.float32)] * 2
            + [pltpu.VMEM((_P, 16), jnp.float32)] * 8
            + [pltpu.VMEM((3 * _P,), jnp.float32)] * 2
            + [pltpu.SemaphoreType.DMA] * 3
        ),
    )
    return f(tab, *ms, *ls, *ws)


def kernel(inputs, ref_img):
    imgt = jnp.transpose(ref_img[0], (1, 2, 0))
    tab = jnp.pad(imgt, ((0, 0), (0, 0), (0, 1))).reshape(_NLINE, 16)
    tab = jax.lax.optimization_barrier(tab)
    outs = _coord_call(inputs)
    ms = outs[:4]
    ls = outs[4:6]
    ws = outs[6:]
    outflat = _sc_call(tab, ms, ls, ws)
    return outflat.reshape(_B, 3, _H, _W)


# speedup vs baseline: 53.1688x; 53.1688x over previous
import math

import jax
import jax.numpy as jnp
from jax import lax
from jax.experimental import pallas as pl
from jax.experimental.pallas import tpu as pltpu
from jax.experimental.pallas import tpu_sc as plsc

_H = 512
_W = 512
_B = 2
_IMH = 2048
_IMW = 2048
_NPOS = _B * _H * _W
_NPIX = _IMH * _IMW
_NLINE = _NPIX // 4
_HW = _H * _W

_NW = 32
_PERW = _NPOS // _NW
_P = 512
_CHUNKS = _PERW // _P

_PI = math.pi
_RB = 32
_BLK = _RB * _W


def _coord_body(in_ref, m0, m1, m2, m3, l0, l1, w0, w1, w2, w3):
    m_refs = [m0, m1, m2, m3]
    l_refs = [l0, l1]
    w_refs = [w0, w1, w2, w3]
    a0 = -in_ref[0, 0]
    a1 = -in_ref[0, 1]
    a2 = -in_ref[0, 2]
    a3 = -in_ref[0, 3]
    ty = jnp.arctan2(a1, a0)
    tx = jnp.arctan2(a3, a2)

    def to_px(t):
        ic = (t + _PI) / (2.0 * _PI)
        ic = -1.0 + 2.0 * ic
        return (ic + 1.0) * 0.5 * (_IMW - 1)

    x = to_px(tx)
    y = to_px(ty)
    x0f = jnp.floor(x)
    y0f = jnp.floor(y)
    wx1 = x - x0f
    wx0 = 1.0 - wx1
    wy1 = y - y0f
    wy0 = 1.0 - wy1
    x0 = jnp.clip(x0f.astype(jnp.int32), 0, _IMW - 1)
    y0 = jnp.clip(y0f.astype(jnp.int32), 0, _IMH - 1)
    x1 = jnp.minimum(x0 + 1, _IMW - 1)
    y1 = jnp.minimum(y0 + 1, _IMH - 1)
    pix = [y0 * _IMW + x0, y0 * _IMW + x1, y1 * _IMW + x0, y1 * _IMW + x1]
    wsv = [wx0 * wy0, wx1 * wy0, wx0 * wy1, wx1 * wy1]
    for c in range(4):
        m_refs[c][...] = (pix[c] >> 2).reshape(_BLK)
        w_refs[c][...] = wsv[c].reshape(_BLK)
    l_refs[0][...] = ((x0 & 3) * 4).reshape(_BLK)
    l_refs[1][...] = ((x1 & 3) * 4).reshape(_BLK)


_spec1d = pl.BlockSpec((_BLK,), lambda b, r: (b * (_H // _RB) + r,))
_coord_call = pl.pallas_call(
    _coord_body,
    grid=(_B, _H // _RB),
    in_specs=[pl.BlockSpec((1, 4, _RB, _W), lambda b, r: (b, 0, r, 0))],
    out_specs=[_spec1d] * 10,
    out_shape=[jax.ShapeDtypeStruct((_NPOS,), jnp.int32)] * 6
              + [jax.ShapeDtypeStruct((_NPOS,), jnp.float32)] * 4,
)


_QPIX = 4096
_PPW = _NPIX // _NW
_BCHUNKS = _PPW // _QPIX


def _build_body(img, tab, pin0, pin1, tout0, tout1, isem, osem):
    pins = [pin0, pin1]
    touts = [tout0, tout1]
    wid = lax.axis_index("s") * 2 + lax.axis_index("c")
    lane4 = lax.iota(jnp.int32, 16) * 4

    def issue_in(t, k):
        pix0 = wid * _PPW + t * _QPIX
        hs = []
        for ch in range(3):
            hs.append(pltpu.async_copy(
                img.at[pl.ds(ch * _NPIX + pix0, _QPIX)],
                pins[k].at[pl.ds(ch * _QPIX, _QPIX)],
                isem))
        return hs

    def interleave(k):
        pin, tout = pins[k], touts[k]

        def j_body(j, carry):
            for ch in range(3):
                v = pin[pl.ds(ch * _QPIX + j * 16, 16)]
                plsc.store_scatter(tout, [lane4 + (j * 64 + ch)], v)
            return carry

        lax.fori_loop(0, _QPIX // 16, j_body, 0, unroll=2)

    def issue_out(t, k):
        pix0 = wid * _PPW + t * _QPIX
        return [pltpu.async_copy(touts[k],
                                 tab.at[pl.ds(pix0 * 4, _QPIX * 4)],
                                 osem)]

    h_i = [None] * (_BCHUNKS + 1)
    h_o = [None] * _BCHUNKS
    h_i[0] = issue_in(0, 0)
    for t in range(_BCHUNKS):
        k = t % 2
        for h in h_i[t]:
            h.wait()
        if t + 1 < _BCHUNKS:
            h_i[t + 1] = issue_in(t + 1, 1 - k)
        if t >= 2:
            for h in h_o[t - 2]:
                h.wait()
        interleave(k)
        h_o[t] = issue_out(t, k)
    for h in h_o[_BCHUNKS - 2]:
        h.wait()
    for h in h_o[_BCHUNKS - 1]:
        h.wait()


def _build_call(img):
    mesh = plsc.VectorSubcoreMesh(core_axis_name="c", subcore_axis_name="s")
    f = pl.kernel(
        _build_body,
        out_type=jax.ShapeDtypeStruct((_NPIX * 4,), jnp.float32),
        mesh=mesh,
        compiler_params=pltpu.CompilerParams(needs_layout_passes=False,
                                             use_tc_tiling_on_sc=False),
        scratch_types=(
            [pltpu.VMEM((3 * _QPIX,), jnp.float32)] * 2
            + [pltpu.VMEM((4 * _QPIX,), jnp.float32)] * 2
            + [pltpu.SemaphoreType.DMA] * 2
        ),
    )
    return f(img)


def _sc_body(tab, m0, m1, m2, m3, l0, l1, w0, w1, w2, w3, out,
             mi_v0, mi_v1, li_v0, li_v1, w_v0, w_v1,
             La0, Lb0, Lc0, Ld0, La1, Lb1, Lc1, Ld1,
             o_v0, o_v1,
             isem, gsem, osem):
    m_args = [m0, m1, m2, m3]
    l_args = [l0, l1]
    w_args = [w0, w1, w2, w3]
    mi_bufs = [mi_v0, mi_v1]
    li_bufs = [li_v0, li_v1]
    w_bufs = [w_v0, w_v1]
    L_bufs = [[La0, Lb0, Lc0, Ld0], [La1, Lb1, Lc1, Ld1]]
    o_bufs = [o_v0, o_v1]
    wid = lax.axis_index("s") * 2 + lax.axis_index("c")
    b = wid // 16
    lane = lax.iota(jnp.int32, 16)

    def issue_idxw(t, k):
        base = wid * _PERW + t * _P
        hs = []
        for c in range(4):
            hs.append(pltpu.async_copy(m_args[c].at[pl.ds(base, _P)],
                                       mi_bufs[k].at[pl.ds(c * _P, _P)],
                                       isem))
            hs.append(pltpu.async_copy(w_args[c].at[pl.ds(base, _P)],
                                       w_bufs[k].at[pl.ds(c * _P, _P)],
                                       isem))
        for c in range(2):
            hs.append(pltpu.async_copy(l_args[c].at[pl.ds(base, _P)],
                                       li_bufs[k].at[pl.ds(c * _P, _P)],
                                       isem))
        return hs

    def issue_gathers(k):
        hs = []
        for c in range(4):
            hs.append(pltpu.async_copy(
                tab.at[mi_bufs[k].at[pl.ds(c * _P, _P)]],
                L_bufs[k][c],
                gsem))
        return hs

    def combine(k):
        w_v, li_v, out_v = w_bufs[k], li_bufs[k], o_bufs[k]
        Ls = L_bufs[k]

        def j_body(j, carry2):
            off = j * 16
            posv = lane + off
            lv = [li_v[pl.ds(0 * _P + off, 16)], li_v[pl.ds(1 * _P + off, 16)]]
            wvs = [w_v[pl.ds(c * _P + off, 16)] for c in range(4)]
            for ch in range(3):
                acc = None
                for c in range(4):
                    lanev = lv[c & 1] + ch
                    val = plsc.load_gather(Ls[c], [posv, lanev])
                    term = wvs[c] * val
                    acc = term if acc is None else acc + term
                out_v[pl.ds(ch * _P + off, 16)] = acc
            return carry2

        lax.fori_loop(0, _P // 16, j_body, 0, unroll=2)

    def issue_outwrite(t, k):
        base = wid * _PERW + t * _P
        inb = base - b * _HW
        hs = []
        for ch in range(3):
            hs.append(pltpu.async_copy(
                o_bufs[k].at[pl.ds(ch * _P, _P)],
                out.at[pl.ds((b * 3 + ch) * _HW + inb, _P)],
                osem))
        return hs

    h_iw = [None] * (_CHUNKS + 2)
    h_g = [None] * _CHUNKS
    h_o = [None] * _CHUNKS

    h_iw[0] = issue_idxw(0, 0)
    for h in h_iw[0]:
        h.wait()
    h_g[0] = issue_gathers(0)
    h_iw[1] = issue_idxw(1, 1)

    for t in range(_CHUNKS):
        k = t % 2
        for h in h_g[t]:
            h.wait()
        if t + 1 < _CHUNKS:
            for h in h_iw[t + 1]:
                h.wait()
            h_g[t + 1] = issue_gathers(1 - k)
        if t >= 2:
            for h in h_o[t - 2]:
                h.wait()
        combine(k)
        h_o[t] = issue_outwrite(t, k)
        if t + 2 < _CHUNKS:
            h_iw[t + 2] = issue_idxw(t + 2, k)
    for h in h_o[_CHUNKS - 2]:
        h.wait()
    for h in h_o[_CHUNKS - 1]:
        h.wait()


def _sc_call(tab, ms, ls, ws):
    mesh = plsc.VectorSubcoreMesh(core_axis_name="c", subcore_axis_name="s")
    f = pl.kernel(
        _sc_body,
        out_type=jax.ShapeDtypeStruct((_B * 3 * _HW,), jnp.float32),
        mesh=mesh,
        compiler_params=pltpu.CompilerParams(needs_layout_passes=False,
                                             use_tc_tiling_on_sc=False),
        scratch_types=(
            [pltpu.VMEM((4 * _P,), jnp.int32)] * 2
            + [pltpu.VMEM((2 * _P,), jnp.int32)] * 2
            + [pltpu.VMEM((4 * _P,), jnp.float32)] * 2
            + [pltpu.VMEM((_P, 16), jnp.float32)] * 8
            + [pltpu.VMEM((3 * _P,), jnp.float32)] * 2
            + [pltpu.SemaphoreType.DMA] * 3
        ),
    )
    return f(tab, *ms, *ls, *ws)


def kernel(inputs, ref_img):
    img = ref_img.reshape(3 * _NPIX)
    tab = _build_call(img).reshape(_NLINE, 16)
    outs = _coord_call(inputs)
    ms = outs[:4]
    ls = outs[4:6]
    ws = outs[6:]
    outflat = _sc_call(tab, ms, ls, ws)
    return outflat.reshape(_B, 3, _H, _W)


# speedup vs baseline: 60.4839x; 1.1376x over previous
import math

import jax
import jax.numpy as jnp
from jax import lax
from jax.experimental import pallas as pl
from jax.experimental.pallas import tpu as pltpu
from jax.experimental.pallas import tpu_sc as plsc

_H = 512
_W = 512
_B = 2
_IMH = 2048
_IMW = 2048
_NPOS = _B * _H * _W
_NPIX = _IMH * _IMW
_NLINE = _NPIX // 4
_HW = _H * _W

_NW = 32
_PERW = _NPOS // _NW
_P = 512
_CHUNKS = _PERW // _P

_PI = math.pi
_RB = 32
_BLK = _RB * _W


def _coord_body(in_ref, m0, m1, m2, m3, l0, l1, w0, w1, w2, w3):
    m_refs = [m0, m1, m2, m3]
    l_refs = [l0, l1]
    w_refs = [w0, w1, w2, w3]
    a0 = -in_ref[0, 0]
    a1 = -in_ref[0, 1]
    a2 = -in_ref[0, 2]
    a3 = -in_ref[0, 3]
    ty = jnp.arctan2(a1, a0)
    tx = jnp.arctan2(a3, a2)

    def to_px(t):
        ic = (t + _PI) / (2.0 * _PI)
        ic = -1.0 + 2.0 * ic
        return (ic + 1.0) * 0.5 * (_IMW - 1)

    x = to_px(tx)
    y = to_px(ty)
    x0f = jnp.floor(x)
    y0f = jnp.floor(y)
    wx1 = x - x0f
    wx0 = 1.0 - wx1
    wy1 = y - y0f
    wy0 = 1.0 - wy1
    x0 = jnp.clip(x0f.astype(jnp.int32), 0, _IMW - 1)
    y0 = jnp.clip(y0f.astype(jnp.int32), 0, _IMH - 1)
    x1 = jnp.minimum(x0 + 1, _IMW - 1)
    y1 = jnp.minimum(y0 + 1, _IMH - 1)
    pix = [y0 * _IMW + x0, y0 * _IMW + x1, y1 * _IMW + x0, y1 * _IMW + x1]
    wsv = [wx0 * wy0, wx1 * wy0, wx0 * wy1, wx1 * wy1]
    for c in range(4):
        m_refs[c][...] = (pix[c] >> 2).reshape(_BLK)
        w_refs[c][...] = wsv[c].reshape(_BLK)
    l_refs[0][...] = ((x0 & 3) * 4).reshape(_BLK)
    l_refs[1][...] = ((x1 & 3) * 4).reshape(_BLK)


_spec1d = pl.BlockSpec((_BLK,), lambda b, r: (b * (_H // _RB) + r,))
_coord_call = pl.pallas_call(
    _coord_body,
    grid=(_B, _H // _RB),
    in_specs=[pl.BlockSpec((1, 4, _RB, _W), lambda b, r: (b, 0, r, 0))],
    out_specs=[_spec1d] * 10,
    out_shape=[jax.ShapeDtypeStruct((_NPOS,), jnp.int32)] * 6
              + [jax.ShapeDtypeStruct((_NPOS,), jnp.float32)] * 4,
)


_BCPW = 1024 // _NW


def _build_body(img, tab, vin0, vin1, tout0, tout1, isem, osem):
    vins = [vin0, vin1]
    touts = [tout0, tout1]
    wid = lax.axis_index("s") * 2 + lax.axis_index("c")
    lane4 = lax.iota(jnp.int32, 16) * 4

    def chunk_coords(t):
        cid = wid * _BCPW + t
        y0 = (cid // 4) * 8
        x0 = (cid % 4) * 512
        return y0, x0

    def issue_in(t, k):
        y0, x0 = chunk_coords(t)
        hs = []
        for ch in range(3):
            hs.append(pltpu.async_copy(
                img.at[0, ch, pl.ds(y0, 8), pl.ds(x0, 512)],
                vins[k].at[ch],
                isem))
        return hs

    def interleave(k):
        vin, tout = vins[k], touts[k]

        def j_body(j, carry):
            r = j // 32
            c0 = (j % 32) * 16
            for ch in range(3):
                v = vin[ch, r, pl.ds(c0, 16)]
                plsc.store_scatter(
                    tout, [lane4 + (r * 2048 + c0 * 4 + ch)], v)
            return carry

        lax.fori_loop(0, 8 * 32, j_body, 0, unroll=2)

    def issue_out(t, k):
        y0, x0 = chunk_coords(t)
        hs = []
        for r in range(8):
            hs.append(pltpu.async_copy(
                touts[k].at[pl.ds(r * 2048, 2048)],
                tab.at[pl.ds(((y0 + r) * _IMW + x0) * 4, 2048)],
                osem))
        return hs

    h_i = [None] * (_BCPW + 1)
    h_o = [None] * _BCPW
    h_i[0] = issue_in(0, 0)
    for t in range(_BCPW):
        k = t % 2
        for h in h_i[t]:
            h.wait()
        if t + 1 < _BCPW:
            h_i[t + 1] = issue_in(t + 1, 1 - k)
        if t >= 2:
            for h in h_o[t - 2]:
                h.wait()
        interleave(k)
        h_o[t] = issue_out(t, k)
    for h in h_o[_BCPW - 2]:
        h.wait()
    for h in h_o[_BCPW - 1]:
        h.wait()


def _build_call(ref_img):
    mesh = plsc.VectorSubcoreMesh(core_axis_name="c", subcore_axis_name="s")
    f = pl.kernel(
        _build_body,
        out_type=jax.ShapeDtypeStruct((_NPIX * 4,), jnp.float32),
        mesh=mesh,
        compiler_params=pltpu.CompilerParams(needs_layout_passes=False,
                                             use_tc_tiling_on_sc=True),
        scratch_types=(
            [pltpu.VMEM((3, 8, 512), jnp.float32)] * 2
            + [pltpu.VMEM((8 * 512 * 4,), jnp.float32)] * 2
            + [pltpu.SemaphoreType.DMA] * 2
        ),
    )
    return f(ref_img)


def _sc_body(tab, m0, m1, m2, m3, l0, l1, w0, w1, w2, w3, out,
             mi_v0, mi_v1, li_v0, li_v1, w_v0, w_v1,
             La0, Lb0, Lc0, Ld0, La1, Lb1, Lc1, Ld1,
             o_v0, o_v1,
             isem, gsem, osem):
    m_args = [m0, m1, m2, m3]
    l_args = [l0, l1]
    w_args = [w0, w1, w2, w3]
    mi_bufs = [mi_v0, mi_v1]
    li_bufs = [li_v0, li_v1]
    w_bufs = [w_v0, w_v1]
    L_bufs = [[La0, Lb0, Lc0, Ld0], [La1, Lb1, Lc1, Ld1]]
    o_bufs = [o_v0, o_v1]
    wid = lax.axis_index("s") * 2 + lax.axis_index("c")
    b = wid // 16
    lane = lax.iota(jnp.int32, 16)

    def issue_idxw(t, k):
        base = wid * _PERW + t * _P
        hs = []
        for c in range(4):
            hs.append(pltpu.async_copy(m_args[c].at[pl.ds(base, _P)],
                                       mi_bufs[k].at[pl.ds(c * _P, _P)],
                                       isem))
            hs.append(pltpu.async_copy(w_args[c].at[pl.ds(base, _P)],
                                       w_bufs[k].at[pl.ds(c * _P, _P)],
                                       isem))
        for c in range(2):
            hs.append(pltpu.async_copy(l_args[c].at[pl.ds(base, _P)],
                                       li_bufs[k].at[pl.ds(c * _P, _P)],
                                       isem))
        return hs

    def issue_gathers(k):
        hs = []
        for c in range(4):
            hs.append(pltpu.async_copy(
                tab.at[mi_bufs[k].at[pl.ds(c * _P, _P)]],
                L_bufs[k][c],
                gsem))
        return hs

    def combine(k):
        w_v, li_v, out_v = w_bufs[k], li_bufs[k], o_bufs[k]
        Ls = L_bufs[k]

        def j_body(j, carry2):
            off = j * 16
            posv = lane + off
            lv = [li_v[pl.ds(0 * _P + off, 16)], li_v[pl.ds(1 * _P + off, 16)]]
            wvs = [w_v[pl.ds(c * _P + off, 16)] for c in range(4)]
            for ch in range(3):
                acc = None
                for c in range(4):
                    lanev = lv[c & 1] + ch
                    val = plsc.load_gather(Ls[c], [posv, lanev])
                    term = wvs[c] * val
                    acc = term if acc is None else acc + term
                out_v[pl.ds(ch * _P + off, 16)] = acc
            return carry2

        lax.fori_loop(0, _P // 16, j_body, 0, unroll=2)

    def issue_outwrite(t, k):
        base = wid * _PERW + t * _P
        inb = base - b * _HW
        hs = []
        for ch in range(3):
            hs.append(pltpu.async_copy(
                o_bufs[k].at[pl.ds(ch * _P, _P)],
                out.at[pl.ds((b * 3 + ch) * _HW + inb, _P)],
                osem))
        return hs

    h_iw = [None] * (_CHUNKS + 2)
    h_g = [None] * _CHUNKS
    h_o = [None] * _CHUNKS

    h_iw[0] = issue_idxw(0, 0)
    for h in h_iw[0]:
        h.wait()
    h_g[0] = issue_gathers(0)
    h_iw[1] = issue_idxw(1, 1)

    for t in range(_CHUNKS):
        k = t % 2
        for h in h_g[t]:
            h.wait()
        if t + 1 < _CHUNKS:
            for h in h_iw[t + 1]:
                h.wait()
            h_g[t + 1] = issue_gathers(1 - k)
        if t >= 2:
            for h in h_o[t - 2]:
                h.wait()
        combine(k)
        h_o[t] = issue_outwrite(t, k)
        if t + 2 < _CHUNKS:
            h_iw[t + 2] = issue_idxw(t + 2, k)
    for h in h_o[_CHUNKS - 2]:
        h.wait()
    for h in h_o[_CHUNKS - 1]:
        h.wait()


def _sc_call(tab, ms, ls, ws):
    mesh = plsc.VectorSubcoreMesh(core_axis_name="c", subcore_axis_name="s")
    f = pl.kernel(
        _sc_body,
        out_type=jax.ShapeDtypeStruct((_B * 3 * _HW,), jnp.float32),
        mesh=mesh,
        compiler_params=pltpu.CompilerParams(needs_layout_passes=False,
                                             use_tc_tiling_on_sc=False),
        scratch_types=(
            [pltpu.VMEM((4 * _P,), jnp.int32)] * 2
            + [pltpu.VMEM((2 * _P,), jnp.int32)] * 2
            + [pltpu.VMEM((4 * _P,), jnp.float32)] * 2
            + [pltpu.VMEM((_P, 16), jnp.float32)] * 8
            + [pltpu.VMEM((3 * _P,), jnp.float32)] * 2
            + [pltpu.SemaphoreType.DMA] * 3
        ),
    )
    return f(tab, *ms, *ls, *ws)


def kernel(inputs, ref_img):
    tab = _build_call(ref_img).reshape(_NLINE, 16)
    outs = _coord_call(inputs)
    ms = outs[:4]
    ls = outs[4:6]
    ws = outs[6:]
    outflat = _sc_call(tab, ms, ls, ws)
    return outflat.reshape(_B, 3, _H, _W)


# speedup vs baseline: 64.0514x; 1.0590x over previous
import math

import jax
import jax.numpy as jnp
from jax import lax
from jax.experimental import pallas as pl
from jax.experimental.pallas import tpu as pltpu
from jax.experimental.pallas import tpu_sc as plsc

_H = 512
_W = 512
_B = 2
_IMH = 2048
_IMW = 2048
_NPOS = _B * _H * _W
_NPIX = _IMH * _IMW
_NLINE = _NPIX // 4
_HW = _H * _W

_NW = 32
_PERW = _NPOS // _NW
_P = 512
_CHUNKS = _PERW // _P

_PI = math.pi
_RB = 32
_BLK = _RB * _W


def _coord_body(in_ref, m0, m1, m2, m3, l0, l1, w0, w1, w2, w3):
    m_refs = [m0, m1, m2, m3]
    l_refs = [l0, l1]
    w_refs = [w0, w1, w2, w3]
    a0 = -in_ref[0, 0]
    a1 = -in_ref[0, 1]
    a2 = -in_ref[0, 2]
    a3 = -in_ref[0, 3]
    ty = jnp.arctan2(a1, a0)
    tx = jnp.arctan2(a3, a2)

    def to_px(t):
        ic = (t + _PI) / (2.0 * _PI)
        ic = -1.0 + 2.0 * ic
        return (ic + 1.0) * 0.5 * (_IMW - 1)

    x = to_px(tx)
    y = to_px(ty)
    x0f = jnp.floor(x)
    y0f = jnp.floor(y)
    wx1 = x - x0f
    wx0 = 1.0 - wx1
    wy1 = y - y0f
    wy0 = 1.0 - wy1
    x0 = jnp.clip(x0f.astype(jnp.int32), 0, _IMW - 1)
    y0 = jnp.clip(y0f.astype(jnp.int32), 0, _IMH - 1)
    x1 = jnp.minimum(x0 + 1, _IMW - 1)
    y1 = jnp.minimum(y0 + 1, _IMH - 1)
    pix = [y0 * _IMW + x0, y0 * _IMW + x1, y1 * _IMW + x0, y1 * _IMW + x1]
    wsv = [wx0 * wy0, wx1 * wy0, wx0 * wy1, wx1 * wy1]
    for c in range(4):
        m_refs[c][...] = (pix[c] >> 2).reshape(_BLK)
        w_refs[c][...] = wsv[c].reshape(_BLK)
    l_refs[0][...] = ((x0 & 3) * 4).reshape(_BLK)
    l_refs[1][...] = ((x1 & 3) * 4).reshape(_BLK)


_spec1d = pl.BlockSpec((_BLK,), lambda b, r: (b * (_H // _RB) + r,))
_coord_call = pl.pallas_call(
    _coord_body,
    grid=(_B, _H // _RB),
    in_specs=[pl.BlockSpec((1, 4, _RB, _W), lambda b, r: (b, 0, r, 0))],
    out_specs=[_spec1d] * 10,
    out_shape=[jax.ShapeDtypeStruct((_NPOS,), jnp.int32)] * 6
              + [jax.ShapeDtypeStruct((_NPOS,), jnp.float32)] * 4,
)


_BCPW = 1024 // _NW


def _build_body(img, tab, vin0, vin1, tout0, tout1,
                isem0, isem1, osem0, osem1):
    isems = [isem0, isem1]
    osems = [osem0, osem1]
    vins = [vin0, vin1]
    touts = [tout0, tout1]
    wid = lax.axis_index("s") * 2 + lax.axis_index("c")
    lane4 = lax.iota(jnp.int32, 16) * 4

    def chunk_coords(t):
        cid = wid * _BCPW + t
        y0 = (cid // 4) * 8
        x0 = (cid % 4) * 512
        return y0, x0

    def issue_in(t, k):
        y0, x0 = chunk_coords(t)
        hs = []
        for ch in range(3):
            hs.append(pltpu.async_copy(
                img.at[0, ch, pl.ds(y0, 8), pl.ds(x0, 512)],
                vins[k].at[ch],
                isems[k]))
        return hs

    def interleave(k):
        vin, tout = vins[k], touts[k]

        def j_body(j, carry):
            r = j // 32
            c0 = (j % 32) * 16
            for ch in range(3):
                v = vin[ch, r, pl.ds(c0, 16)]
                plsc.store_scatter(
                    tout, [lane4 + (r * 2048 + c0 * 4 + ch)], v)
            return carry

        lax.fori_loop(0, 8 * 32, j_body, 0, unroll=2)

    def issue_out(t, k):
        y0, x0 = chunk_coords(t)
        hs = []
        for r in range(8):
            hs.append(pltpu.async_copy(
                touts[k].at[pl.ds(r * 2048, 2048)],
                tab.at[pl.ds(((y0 + r) * _IMW + x0) * 4, 2048)],
                osems[k]))
        return hs

    h_i = [None] * (_BCPW + 1)
    h_o = [None] * _BCPW
    h_i[0] = issue_in(0, 0)
    for t in range(_BCPW):
        k = t % 2
        if t + 1 < _BCPW:
            h_i[t + 1] = issue_in(t + 1, 1 - k)
        for h in h_i[t]:
            h.wait()
        if t >= 2:
            for h in h_o[t - 2]:
                h.wait()
        interleave(k)
        h_o[t] = issue_out(t, k)
    for h in h_o[_BCPW - 2]:
        h.wait()
    for h in h_o[_BCPW - 1]:
        h.wait()


def _build_call(ref_img):
    mesh = plsc.VectorSubcoreMesh(core_axis_name="c", subcore_axis_name="s")
    f = pl.kernel(
        _build_body,
        out_type=jax.ShapeDtypeStruct((_NPIX * 4,), jnp.float32),
        mesh=mesh,
        compiler_params=pltpu.CompilerParams(needs_layout_passes=False,
                                             use_tc_tiling_on_sc=True),
        scratch_types=(
            [pltpu.VMEM((3, 8, 512), jnp.float32)] * 2
            + [pltpu.VMEM((8 * 512 * 4,), jnp.float32)] * 2
            + [pltpu.SemaphoreType.DMA] * 4
        ),
    )
    return f(ref_img)


def _sc_body(tab, m0, m1, m2, m3, l0, l1, w0, w1, w2, w3, out,
             mi_v0, mi_v1, li_v0, li_v1, w_v0, w_v1,
             La0, Lb0, Lc0, Ld0, La1, Lb1, Lc1, Ld1,
             o_v0, o_v1,
             isem0, isem1, gsem0, gsem1, osem0, osem1):
    isems = [isem0, isem1]
    gsems = [gsem0, gsem1]
    osems = [osem0, osem1]
    m_args = [m0, m1, m2, m3]
    l_args = [l0, l1]
    w_args = [w0, w1, w2, w3]
    mi_bufs = [mi_v0, mi_v1]
    li_bufs = [li_v0, li_v1]
    w_bufs = [w_v0, w_v1]
    L_bufs = [[La0, Lb0, Lc0, Ld0], [La1, Lb1, Lc1, Ld1]]
    o_bufs = [o_v0, o_v1]
    wid = lax.axis_index("s") * 2 + lax.axis_index("c")
    b = wid // 16
    lane = lax.iota(jnp.int32, 16)

    def issue_idxw(t, k):
        base = wid * _PERW + t * _P
        hs = []
        for c in range(4):
            hs.append(pltpu.async_copy(m_args[c].at[pl.ds(base, _P)],
                                       mi_bufs[k].at[pl.ds(c * _P, _P)],
                                       isems[k]))
            hs.append(pltpu.async_copy(w_args[c].at[pl.ds(base, _P)],
                                       w_bufs[k].at[pl.ds(c * _P, _P)],
                                       isems[k]))
        for c in range(2):
            hs.append(pltpu.async_copy(l_args[c].at[pl.ds(base, _P)],
                                       li_bufs[k].at[pl.ds(c * _P, _P)],
                                       isems[k]))
        return hs

    def issue_gathers(k):
        hs = []
        for c in range(4):
            hs.append(pltpu.async_copy(
                tab.at[mi_bufs[k].at[pl.ds(c * _P, _P)]],
                L_bufs[k][c],
                gsems[k]))
        return hs

    def combine(k):
        w_v, li_v, out_v = w_bufs[k], li_bufs[k], o_bufs[k]
        Ls = L_bufs[k]

        def j_body(j, carry2):
            off = j * 16
            posv = lane + off
            lv = [li_v[pl.ds(0 * _P + off, 16)], li_v[pl.ds(1 * _P + off, 16)]]
            wvs = [w_v[pl.ds(c * _P + off, 16)] for c in range(4)]
            for ch in range(3):
                acc = None
                for c in range(4):
                    lanev = lv[c & 1] + ch
                    val = plsc.load_gather(Ls[c], [posv, lanev])
                    term = wvs[c] * val
                    acc = term if acc is None else acc + term
                out_v[pl.ds(ch * _P + off, 16)] = acc
            return carry2

        lax.fori_loop(0, _P // 16, j_body, 0, unroll=2)

    def issue_outwrite(t, k):
        base = wid * _PERW + t * _P
        inb = base - b * _HW
        hs = []
        for ch in range(3):
            hs.append(pltpu.async_copy(
                o_bufs[k].at[pl.ds(ch * _P, _P)],
                out.at[pl.ds((b * 3 + ch) * _HW + inb, _P)],
                osems[k]))
        return hs

    h_iw = [None] * (_CHUNKS + 2)
    h_g = [None] * _CHUNKS
    h_o = [None] * _CHUNKS

    h_iw[0] = issue_idxw(0, 0)
    for h in h_iw[0]:
        h.wait()
    h_g[0] = issue_gathers(0)
    h_iw[1] = issue_idxw(1, 1)

    for t in range(_CHUNKS):
        k = t % 2
        if t + 1 < _CHUNKS:
            for h in h_iw[t + 1]:
                h.wait()
            h_g[t + 1] = issue_gathers(1 - k)
        for h in h_g[t]:
            h.wait()
        if t >= 2:
            for h in h_o[t - 2]:
                h.wait()
        combine(k)
        h_o[t] = issue_outwrite(t, k)
        if t + 2 < _CHUNKS:
            h_iw[t + 2] = issue_idxw(t + 2, k)
    for h in h_o[_CHUNKS - 2]:
        h.wait()
    for h in h_o[_CHUNKS - 1]:
        h.wait()


def _sc_call(tab, ms, ls, ws):
    mesh = plsc.VectorSubcoreMesh(core_axis_name="c", subcore_axis_name="s")
    f = pl.kernel(
        _sc_body,
        out_type=jax.ShapeDtypeStruct((_B * 3 * _HW,), jnp.float32),
        mesh=mesh,
        compiler_params=pltpu.CompilerParams(needs_layout_passes=False,
                                             use_tc_tiling_on_sc=False),
        scratch_types=(
            [pltpu.VMEM((4 * _P,), jnp.int32)] * 2
            + [pltpu.VMEM((2 * _P,), jnp.int32)] * 2
            + [pltpu.VMEM((4 * _P,), jnp.float32)] * 2
            + [pltpu.VMEM((_P, 16), jnp.float32)] * 8
            + [pltpu.VMEM((3 * _P,), jnp.float32)] * 2
            + [pltpu.SemaphoreType.DMA] * 6
        ),
    )
    return f(tab, *ms, *ls, *ws)


def kernel(inputs, ref_img):
    tab = _build_call(ref_img).reshape(_NLINE, 16)
    outs = _coord_call(inputs)
    ms = outs[:4]
    ls = outs[4:6]
    ws = outs[6:]
    outflat = _sc_call(tab, ms, ls, ws)
    return outflat.reshape(_B, 3, _H, _W)


# speedup vs baseline: 64.9904x; 1.0147x over previous
import math

import jax
import jax.numpy as jnp
from jax import lax
from jax.experimental import pallas as pl
from jax.experimental.pallas import tpu as pltpu
from jax.experimental.pallas import tpu_sc as plsc

_H = 512
_W = 512
_B = 2
_IMH = 2048
_IMW = 2048
_NPOS = _B * _H * _W
_NPIX = _IMH * _IMW
_NLINE = _NPIX // 4
_HW = _H * _W

_NW = 32
_PERW = _NPOS // _NW
_P = 512
_CHUNKS = _PERW // _P

_PI = math.pi
_RB = 32
_BLK = _RB * _W


def _coord_body(in_ref, m0, m1, m2, m3, l0, l1, w0, w1, w2, w3):
    m_refs = [m0, m1, m2, m3]
    l_refs = [l0, l1]
    w_refs = [w0, w1, w2, w3]
    a0 = -in_ref[0, 0]
    a1 = -in_ref[0, 1]
    a2 = -in_ref[0, 2]
    a3 = -in_ref[0, 3]
    ty = jnp.arctan2(a1, a0)
    tx = jnp.arctan2(a3, a2)

    def to_px(t):
        ic = (t + _PI) / (2.0 * _PI)
        ic = -1.0 + 2.0 * ic
        return (ic + 1.0) * 0.5 * (_IMW - 1)

    x = to_px(tx)
    y = to_px(ty)
    x0f = jnp.floor(x)
    y0f = jnp.floor(y)
    wx1 = x - x0f
    wx0 = 1.0 - wx1
    wy1 = y - y0f
    wy0 = 1.0 - wy1
    x0 = jnp.clip(x0f.astype(jnp.int32), 0, _IMW - 1)
    y0 = jnp.clip(y0f.astype(jnp.int32), 0, _IMH - 1)
    x1 = jnp.minimum(x0 + 1, _IMW - 1)
    y1 = jnp.minimum(y0 + 1, _IMH - 1)
    pix = [y0 * _IMW + x0, y0 * _IMW + x1, y1 * _IMW + x0, y1 * _IMW + x1]
    wsv = [wx0 * wy0, wx1 * wy0, wx0 * wy1, wx1 * wy1]
    for c in range(4):
        m_refs[c][...] = (pix[c] >> 2).reshape(_BLK)
        w_refs[c][...] = wsv[c].reshape(_BLK)
    l_refs[0][...] = ((x0 & 3) * 4).reshape(_BLK)
    l_refs[1][...] = ((x1 & 3) * 4).reshape(_BLK)


_spec1d = pl.BlockSpec((_BLK,), lambda b, r: (b * (_H // _RB) + r,))
_coord_call = pl.pallas_call(
    _coord_body,
    grid=(_B, _H // _RB),
    in_specs=[pl.BlockSpec((1, 4, _RB, _W), lambda b, r: (b, 0, r, 0))],
    out_specs=[_spec1d] * 10,
    out_shape=[jax.ShapeDtypeStruct((_NPOS,), jnp.int32)] * 6
              + [jax.ShapeDtypeStruct((_NPOS,), jnp.float32)] * 4,
)


_BCPW = 512 // _NW


def _build_body(img, tab, vin0, vin1, tout0, tout1,
                isem0, isem1, osem0, osem1):
    isems = [isem0, isem1]
    osems = [osem0, osem1]
    vins = [vin0, vin1]
    touts = [tout0, tout1]
    wid = lax.axis_index("s") * 2 + lax.axis_index("c")
    lane4 = lax.iota(jnp.int32, 16) * 4

    def chunk_coords(t):
        cid = wid * _BCPW + t
        y0 = (cid // 2) * 8
        x0 = (cid % 2) * 1024
        return y0, x0

    def issue_in(t, k):
        y0, x0 = chunk_coords(t)
        hs = []
        for ch in range(3):
            hs.append(pltpu.async_copy(
                img.at[0, ch, pl.ds(y0, 8), pl.ds(x0, 1024)],
                vins[k].at[ch],
                isems[k]))
        return hs

    def interleave(k):
        vin, tout = vins[k], touts[k]

        def j_body(j, carry):
            r = j // 64
            c0 = (j % 64) * 16
            for ch in range(3):
                v = vin[ch, r, pl.ds(c0, 16)]
                plsc.store_scatter(
                    tout, [lane4 + (r * 4096 + c0 * 4 + ch)], v)
            return carry

        lax.fori_loop(0, 8 * 64, j_body, 0, unroll=2)

    def issue_out(t, k):
        y0, x0 = chunk_coords(t)
        hs = []
        for r in range(8):
            hs.append(pltpu.async_copy(
                touts[k].at[pl.ds(r * 4096, 4096)],
                tab.at[pl.ds(((y0 + r) * _IMW + x0) * 4, 4096)],
                osems[k]))
        return hs

    h_i = [None] * (_BCPW + 1)
    h_o = [None] * _BCPW
    h_i[0] = issue_in(0, 0)
    for t in range(_BCPW):
        k = t % 2
        if t + 1 < _BCPW:
            h_i[t + 1] = issue_in(t + 1, 1 - k)
        for h in h_i[t]:
            h.wait()
        if t >= 2:
            for h in h_o[t - 2]:
                h.wait()
        interleave(k)
        h_o[t] = issue_out(t, k)
    for h in h_o[_BCPW - 2]:
        h.wait()
    for h in h_o[_BCPW - 1]:
        h.wait()


def _build_call(ref_img):
    mesh = plsc.VectorSubcoreMesh(core_axis_name="c", subcore_axis_name="s")
    f = pl.kernel(
        _build_body,
        out_type=jax.ShapeDtypeStruct((_NPIX * 4,), jnp.float32),
        mesh=mesh,
        compiler_params=pltpu.CompilerParams(needs_layout_passes=False,
                                             use_tc_tiling_on_sc=True),
        scratch_types=(
            [pltpu.VMEM((3, 8, 1024), jnp.float32)] * 2
            + [pltpu.VMEM((8 * 1024 * 4,), jnp.float32)] * 2
            + [pltpu.SemaphoreType.DMA] * 4
        ),
    )
    return f(ref_img)


def _sc_body(tab, m0, m1, m2, m3, l0, l1, w0, w1, w2, w3, out,
             mi_v0, mi_v1, li_v0, li_v1, w_v0, w_v1,
             La0, Lb0, Lc0, Ld0, La1, Lb1, Lc1, Ld1,
             o_v0, o_v1,
             isem0, isem1, gsem0, gsem1, osem0, osem1):
    isems = [isem0, isem1]
    gsems = [gsem0, gsem1]
    osems = [osem0, osem1]
    m_args = [m0, m1, m2, m3]
    l_args = [l0, l1]
    w_args = [w0, w1, w2, w3]
    mi_bufs = [mi_v0, mi_v1]
    li_bufs = [li_v0, li_v1]
    w_bufs = [w_v0, w_v1]
    L_bufs = [[La0, Lb0, Lc0, Ld0], [La1, Lb1, Lc1, Ld1]]
    o_bufs = [o_v0, o_v1]
    wid = lax.axis_index("s") * 2 + lax.axis_index("c")
    b = wid // 16
    lane = lax.iota(jnp.int32, 16)

    def issue_idxw(t, k):
        base = wid * _PERW + t * _P
        hs = []
        for c in range(4):
            hs.append(pltpu.async_copy(m_args[c].at[pl.ds(base, _P)],
                                       mi_bufs[k].at[pl.ds(c * _P, _P)],
                                       isems[k]))
            hs.append(pltpu.async_copy(w_args[c].at[pl.ds(base, _P)],
                                       w_bufs[k].at[pl.ds(c * _P, _P)],
                                       isems[k]))
        for c in range(2):
            hs.append(pltpu.async_copy(l_args[c].at[pl.ds(base, _P)],
                                       li_bufs[k].at[pl.ds(c * _P, _P)],
                                       isems[k]))
        return hs

    def issue_gathers(k):
        hs = []
        for c in range(4):
            hs.append(pltpu.async_copy(
                tab.at[mi_bufs[k].at[pl.ds(c * _P, _P)]],
                L_bufs[k][c],
                gsems[k]))
        return hs

    def combine(k):
        w_v, li_v, out_v = w_bufs[k], li_bufs[k], o_bufs[k]
        Ls = L_bufs[k]

        def j_body(j, carry2):
            off = j * 16
            posv = lane + off
            lv = [li_v[pl.ds(0 * _P + off, 16)], li_v[pl.ds(1 * _P + off, 16)]]
            wvs = [w_v[pl.ds(c * _P + off, 16)] for c in range(4)]
            for ch in range(3):
                acc = None
                for c in range(4):
                    lanev = lv[c & 1] + ch
                    val = plsc.load_gather(Ls[c], [posv, lanev])
                    term = wvs[c] * val
                    acc = term if acc is None else acc + term
                out_v[pl.ds(ch * _P + off, 16)] = acc
            return carry2

        lax.fori_loop(0, _P // 16, j_body, 0, unroll=2)

    def issue_outwrite(t, k):
        base = wid * _PERW + t * _P
        inb = base - b * _HW
        hs = []
        for ch in range(3):
            hs.append(pltpu.async_copy(
                o_bufs[k].at[pl.ds(ch * _P, _P)],
                out.at[pl.ds((b * 3 + ch) * _HW + inb, _P)],
                osems[k]))
        return hs

    h_iw = [None] * (_CHUNKS + 2)
    h_g = [None] * _CHUNKS
    h_o = [None] * _CHUNKS

    h_iw[0] = issue_idxw(0, 0)
    for h in h_iw[0]:
        h.wait()
    h_g[0] = issue_gathers(0)
    h_iw[1] = issue_idxw(1, 1)

    for t in range(_CHUNKS):
        k = t % 2
        if t + 1 < _CHUNKS:
            for h in h_iw[t + 1]:
                h.wait()
            h_g[t + 1] = issue_gathers(1 - k)
        for h in h_g[t]:
            h.wait()
        if t >= 2:
            for h in h_o[t - 2]:
                h.wait()
        combine(k)
        h_o[t] = issue_outwrite(t, k)
        if t + 2 < _CHUNKS:
            h_iw[t + 2] = issue_idxw(t + 2, k)
    for h in h_o[_CHUNKS - 2]:
        h.wait()
    for h in h_o[_CHUNKS - 1]:
        h.wait()


def _sc_call(tab, ms, ls, ws):
    mesh = plsc.VectorSubcoreMesh(core_axis_name="c", subcore_axis_name="s")
    f = pl.kernel(
        _sc_body,
        out_type=jax.ShapeDtypeStruct((_B * 3 * _HW,), jnp.float32),
        mesh=mesh,
        compiler_params=pltpu.CompilerParams(needs_layout_passes=False,
                                             use_tc_tiling_on_sc=False),
        scratch_types=(
            [pltpu.VMEM((4 * _P,), jnp.int32)] * 2
            + [pltpu.VMEM((2 * _P,), jnp.int32)] * 2
            + [pltpu.VMEM((4 * _P,), jnp.float32)] * 2
            + [pltpu.VMEM((_P, 16), jnp.float32)] * 8
            + [pltpu.VMEM((3 * _P,), jnp.float32)] * 2
            + [pltpu.SemaphoreType.DMA] * 6
        ),
    )
    return f(tab, *ms, *ls, *ws)


def kernel(inputs, ref_img):
    tab = _build_call(ref_img).reshape(_NLINE, 16)
    outs = _coord_call(inputs)
    ms = outs[:4]
    ls = outs[4:6]
    ws = outs[6:]
    outflat = _sc_call(tab, ms, ls, ws)
    return outflat.reshape(_B, 3, _H, _W)


# speedup vs baseline: 65.0916x; 1.0016x over previous
import math

import jax
import jax.numpy as jnp
from jax import lax
from jax.experimental import pallas as pl
from jax.experimental.pallas import tpu as pltpu
from jax.experimental.pallas import tpu_sc as plsc

_H = 512
_W = 512
_B = 2
_IMH = 2048
_IMW = 2048
_NPOS = _B * _H * _W
_NPIX = _IMH * _IMW
_NLINE = _NPIX // 4
_HW = _H * _W

_NW = 32
_PERW = _NPOS // _NW
_P = 512
_CHUNKS = _PERW // _P

_PI = math.pi
_RB = 32
_BLK = _RB * _W


def _coord_body(in_ref, m0_ref, lf_ref, wx_ref, wy_ref):
    a0 = -in_ref[0, 0]
    a1 = -in_ref[0, 1]
    a2 = -in_ref[0, 2]
    a3 = -in_ref[0, 3]
    ty = jnp.arctan2(a1, a0)
    tx = jnp.arctan2(a3, a2)

    def to_px(t):
        ic = (t + _PI) / (2.0 * _PI)
        ic = -1.0 + 2.0 * ic
        return (ic + 1.0) * 0.5 * (_IMW - 1)

    x = to_px(tx)
    y = to_px(ty)
    x0f = jnp.floor(x)
    y0f = jnp.floor(y)
    wx1 = x - x0f
    wx0 = 1.0 - wx1
    wy1 = y - y0f
    wy0 = 1.0 - wy1
    x0 = jnp.clip(x0f.astype(jnp.int32), 0, _IMW - 1)
    y0 = jnp.clip(y0f.astype(jnp.int32), 0, _IMH - 1)
    x1 = jnp.minimum(x0 + 1, _IMW - 1)
    y1 = jnp.minimum(y0 + 1, _IMH - 1)
    xc = ((x0 & 3) == 3) & (x1 != x0)
    yc = y1 != y0
    lf = ((x0 & 3) * 4 + xc.astype(jnp.int32) * 16
          + yc.astype(jnp.int32) * 32)
    m0_ref[...] = ((y0 * _IMW + x0) >> 2).reshape(_BLK)
    lf_ref[...] = lf.reshape(_BLK)
    wx_ref[...] = wx1.reshape(_BLK)
    wy_ref[...] = wy1.reshape(_BLK)


_spec1d = pl.BlockSpec((_BLK,), lambda b, r: (b * (_H // _RB) + r,))
_coord_call = pl.pallas_call(
    _coord_body,
    grid=(_B, _H // _RB),
    in_specs=[pl.BlockSpec((1, 4, _RB, _W), lambda b, r: (b, 0, r, 0))],
    out_specs=[_spec1d] * 4,
    out_shape=[jax.ShapeDtypeStruct((_NPOS,), jnp.int32)] * 2
              + [jax.ShapeDtypeStruct((_NPOS,), jnp.float32)] * 2,
)


_BCPW = 512 // _NW


def _build_body(img, tab, vin0, vin1, tout0, tout1,
                isem0, isem1, osem0, osem1):
    isems = [isem0, isem1]
    osems = [osem0, osem1]
    vins = [vin0, vin1]
    touts = [tout0, tout1]
    wid = lax.axis_index("s") * 2 + lax.axis_index("c")
    lane4 = lax.iota(jnp.int32, 16) * 4

    def chunk_coords(t):
        cid = wid * _BCPW + t
        y0 = (cid // 2) * 8
        x0 = (cid % 2) * 1024
        return y0, x0

    def issue_in(t, k):
        y0, x0 = chunk_coords(t)
        hs = []
        for ch in range(3):
            hs.append(pltpu.async_copy(
                img.at[0, ch, pl.ds(y0, 8), pl.ds(x0, 1024)],
                vins[k].at[ch],
                isems[k]))
        return hs

    def interleave(k):
        vin, tout = vins[k], touts[k]

        def j_body(j, carry):
            r = j // 64
            c0 = (j % 64) * 16
            for ch in range(3):
                v = vin[ch, r, pl.ds(c0, 16)]
                plsc.store_scatter(
                    tout, [lane4 + (r * 4096 + c0 * 4 + ch)], v)
            return carry

        lax.fori_loop(0, 8 * 64, j_body, 0, unroll=2)

    def issue_out(t, k):
        y0, x0 = chunk_coords(t)
        hs = []
        for r in range(8):
            hs.append(pltpu.async_copy(
                touts[k].at[pl.ds(r * 4096, 4096)],
                tab.at[pl.ds(((y0 + r) * _IMW + x0) * 4, 4096)],
                osems[k]))
        return hs

    h_i = [None] * (_BCPW + 1)
    h_o = [None] * _BCPW
    h_i[0] = issue_in(0, 0)
    for t in range(_BCPW):
        k = t % 2
        if t + 1 < _BCPW:
            h_i[t + 1] = issue_in(t + 1, 1 - k)
        for h in h_i[t]:
            h.wait()
        if t >= 2:
            for h in h_o[t - 2]:
                h.wait()
        interleave(k)
        h_o[t] = issue_out(t, k)
    for h in h_o[_BCPW - 2]:
        h.wait()
    for h in h_o[_BCPW - 1]:
        h.wait()


def _build_call(ref_img):
    mesh = plsc.VectorSubcoreMesh(core_axis_name="c", subcore_axis_name="s")
    f = pl.kernel(
        _build_body,
        out_type=jax.ShapeDtypeStruct((_NPIX * 4,), jnp.float32),
        mesh=mesh,
        compiler_params=pltpu.CompilerParams(needs_layout_passes=False,
                                             use_tc_tiling_on_sc=True),
        scratch_types=(
            [pltpu.VMEM((3, 8, 1024), jnp.float32)] * 2
            + [pltpu.VMEM((8 * 1024 * 4,), jnp.float32)] * 2
            + [pltpu.SemaphoreType.DMA] * 4
        ),
    )
    return f(ref_img)


def _sc_body(tab, m0a, lfa, wxa, wya, out,
             st_v0, st_v1, mi_v0, mi_v1,
             La0, Lb0, Lc0, Ld0, La1, Lb1, Lc1, Ld1,
             o_v0, o_v1,
             isem0, isem1, gsem0, gsem1, osem0, osem1):
    isems = [isem0, isem1]
    gsems = [gsem0, gsem1]
    osems = [osem0, osem1]
    st_args = [m0a, lfa, wxa, wya]
    st_bufs = [st_v0, st_v1]
    mi_bufs = [mi_v0, mi_v1]
    L_bufs = [[La0, Lb0, Lc0, Ld0], [La1, Lb1, Lc1, Ld1]]
    o_bufs = [o_v0, o_v1]
    wid = lax.axis_index("s") * 2 + lax.axis_index("c")
    b = wid // 16
    lane = lax.iota(jnp.int32, 16)

    def issue_idxw(t, k):
        base = wid * _PERW + t * _P
        hs = []
        for c in range(4):
            hs.append(pltpu.async_copy(st_args[c].at[pl.ds(base, _P)],
                                       st_bufs[k].at[pl.ds(c * _P, _P)],
                                       isems[k]))
        return hs

    def derive(k):
        st_v, mi_v = st_bufs[k], mi_bufs[k]

        def j_body(j, carry):
            off = j * 16
            m0v = st_v[pl.ds(off, 16)]
            lfv = st_v[pl.ds(_P + off, 16)]
            xc = (lfv >> 4) & 1
            yoff = ((lfv >> 5) & 1) * 512
            m1v = m0v + xc
            mi_v[pl.ds(off, 16)] = m0v
            mi_v[pl.ds(_P + off, 16)] = m1v
            mi_v[pl.ds(2 * _P + off, 16)] = m0v + yoff
            mi_v[pl.ds(3 * _P + off, 16)] = m1v + yoff
            return carry

        lax.fori_loop(0, _P // 16, j_body, 0, unroll=2)

    def issue_gathers(k):
        hs = []
        for c in range(4):
            hs.append(pltpu.async_copy(
                tab.at[mi_bufs[k].at[pl.ds(c * _P, _P)]],
                L_bufs[k][c],
                gsems[k]))
        return hs

    def combine(k):
        st_v, out_v = st_bufs[k], o_bufs[k]
        Ls = L_bufs[k]

        def j_body(j, carry2):
            off = j * 16
            posv = lane + off
            lfv = st_v[pl.ds(_P + off, 16)]
            l0v = lfv & 15
            xc1 = ((lfv >> 4) & 1) == 1
            l1v = jnp.where(xc1, 0, jnp.minimum(l0v + 4, 12))
            lv = [l0v, l1v]
            wxv = plsc.bitcast(st_v[pl.ds(2 * _P + off, 16)], jnp.float32)
            wyv = plsc.bitcast(st_v[pl.ds(3 * _P + off, 16)], jnp.float32)
            wx0 = 1.0 - wxv
            wy0 = 1.0 - wyv
            wvs = [wx0 * wy0, wxv * wy0, wx0 * wyv, wxv * wyv]
            for ch in range(3):
                acc = None
                for c in range(4):
                    lanev = lv[c & 1] + ch
                    val = plsc.load_gather(Ls[c], [posv, lanev])
                    term = wvs[c] * val
                    acc = term if acc is None else acc + term
                out_v[pl.ds(ch * _P + off, 16)] = acc
            return carry2

        lax.fori_loop(0, _P // 16, j_body, 0, unroll=2)

    def issue_outwrite(t, k):
        base = wid * _PERW + t * _P
        inb = base - b * _HW
        hs = []
        for ch in range(3):
            hs.append(pltpu.async_copy(
                o_bufs[k].at[pl.ds(ch * _P, _P)],
                out.at[pl.ds((b * 3 + ch) * _HW + inb, _P)],
                osems[k]))
        return hs

    h_iw = [None] * (_CHUNKS + 2)
    h_g = [None] * _CHUNKS
    h_o = [None] * _CHUNKS

    h_iw[0] = issue_idxw(0, 0)
    for h in h_iw[0]:
        h.wait()
    derive(0)
    h_g[0] = issue_gathers(0)
    h_iw[1] = issue_idxw(1, 1)

    for t in range(_CHUNKS):
        k = t % 2
        if t + 1 < _CHUNKS:
            for h in h_iw[t + 1]:
                h.wait()
            derive(1 - k)
            h_g[t + 1] = issue_gathers(1 - k)
        for h in h_g[t]:
            h.wait()
        if t >= 2:
            for h in h_o[t - 2]:
                h.wait()
        combine(k)
        h_o[t] = issue_outwrite(t, k)
        if t + 2 < _CHUNKS:
            h_iw[t + 2] = issue_idxw(t + 2, k)
    for h in h_o[_CHUNKS - 2]:
        h.wait()
    for h in h_o[_CHUNKS - 1]:
        h.wait()


def _sc_call(tab, m0a, lfa, wxa, wya):
    mesh = plsc.VectorSubcoreMesh(core_axis_name="c", subcore_axis_name="s")
    f = pl.kernel(
        _sc_body,
        out_type=jax.ShapeDtypeStruct((_B * 3 * _HW,), jnp.float32),
        mesh=mesh,
        compiler_params=pltpu.CompilerParams(needs_layout_passes=False,
                                             use_tc_tiling_on_sc=False),
        scratch_types=(
            [pltpu.VMEM((4 * _P,), jnp.int32)] * 4
            + [pltpu.VMEM((_P, 16), jnp.float32)] * 8
            + [pltpu.VMEM((3 * _P,), jnp.float32)] * 2
            + [pltpu.SemaphoreType.DMA] * 6
        ),
    )
    return f(tab, m0a, lfa, wxa, wya)


def kernel(inputs, ref_img):
    tab = _build_call(ref_img).reshape(_NLINE, 16)
    m0a, lfa, wxa, wya = _coord_call(inputs)
    wxa = jax.lax.bitcast_convert_type(wxa, jnp.int32)
    wya = jax.lax.bitcast_convert_type(wya, jnp.int32)
    outflat = _sc_call(tab, m0a, lfa, wxa, wya)
    return outflat.reshape(_B, 3, _H, _W)
